# baseline (speedup 1.0000x reference)
import jax
import jax.numpy as jnp
import numpy as np
from jax.experimental import pallas as pl
from jax.experimental.pallas import tpu as pltpu

IN_FEATS = 2048
OUT = 10
CHUNKS = 16
THRESH = 0.05
CHUNK_DIM = IN_FEATS // CHUNKS
N_TOK = 16384

GRP = 16
WIDE = CHUNKS * GRP
BLK = 2048
NSTEPS = N_TOK // BLK


_DN0 = (((0,), (0,)), ((), ()))


def _prep_kernel(ew_ref, rw_ref, eb_ref, rb_ref, m10_ref, mr_ref,
                 w_ref, b_ref):
    lane = jax.lax.broadcasted_iota(jnp.int32, (1, WIDE), 1)
    rowc = jax.lax.broadcasted_iota(jnp.int32, (IN_FEATS, 1), 0) // CHUNK_DIM
    spread = jax.lax.dot_general(ew_ref[:], m10_ref[:], _DN0,
                                 preferred_element_type=jnp.float32)
    w_ref[:] = (jnp.where((lane // GRP) == rowc, spread, 0.0)
                + jax.lax.dot_general(rw_ref[:], mr_ref[:], _DN0,
                                      preferred_element_type=jnp.float32))
    ebs = jnp.dot(eb_ref[:], m10_ref[:], preferred_element_type=jnp.float32)
    crow = jax.lax.broadcasted_iota(jnp.int32, (CHUNKS, 1), 0)
    b_ref[:] = (jnp.sum(jnp.where((lane // GRP) == crow, ebs, 0.0),
                        axis=0, keepdims=True)
                + jnp.dot(rb_ref[:], mr_ref[:],
                          preferred_element_type=jnp.float32))


def _main_kernel(x_ref, w_ref, b_ref, rout_ref, smat_ref,
                 outpre_ref, acts_ref):
    grp = jax.lax.broadcasted_iota(jnp.int32, (1, WIDE), 1) // GRP
    ci = jax.lax.broadcasted_iota(jnp.int32, (1, CHUNKS), 1)

    acc = jnp.dot(x_ref[:], w_ref[:], preferred_element_type=jnp.float32)
    acc = acc + b_ref[:]

    l16 = jnp.dot(acc, smat_ref[:], precision=jax.lax.Precision.HIGHEST,
                  preferred_element_type=jnp.float32)
    m = jnp.max(l16, axis=1, keepdims=True)
    e = jnp.exp(l16 - m)
    s = jnp.sum(e, axis=1, keepdims=True)
    rcp = 1.0 / s
    c1 = jnp.min(jnp.where(l16 == m, ci, CHUNKS), axis=1, keepdims=True)
    lm2 = jnp.where(ci == c1, -jnp.inf, l16)
    m2 = jnp.max(lm2, axis=1, keepdims=True)
    c2 = jnp.min(jnp.where(lm2 == m2, ci, CHUNKS), axis=1, keepdims=True)
    v2 = jnp.exp(m2 - m) * rcp

    gates = (jnp.where(grp == c1, rcp, 0.0)
             + jnp.where(grp == c2, v2, 0.0))

    outpre_ref[:] = jnp.dot(gates * acc, rout_ref[:],
                            preferred_element_type=jnp.float32)

    acts_ref[:] = jnp.sum(jnp.abs(acc), axis=0, keepdims=True)[None]


def _head_kernel(outpre_ref, acts_ref, rchunk_ref, qw_ref, qb_ref, o_ref):
    colsum = jnp.sum(acts_ref[:], axis=0)
    acts16 = jnp.dot(colsum, rchunk_ref[:],
                     preferred_element_type=jnp.float32)
    scalar_act = jnp.max(acts16) * (1.0 / (N_TOK * OUT))
    qw = jnp.pad(qw_ref[:], ((0, GRP - OUT), (0, GRP - OUT)))
    mean_abs = jnp.sum(jnp.abs(qw)) * (1.0 / (OUT * OUT))
    wq = jnp.where(scalar_act > THRESH, qw, jnp.sign(qw) * mean_abs)
    res_t = jax.lax.dot_general(wq, outpre_ref[:], (((0,), (1,)), ((), ())),
                                preferred_element_type=jnp.float32)
    res_t = res_t + jnp.pad(qb_ref[:], ((0, GRP - OUT), (0, 0)))
    o_ref[:] = res_t[:OUT, :]


def kernel(x, router_w, router_b, expert_w, expert_b, quant_w, quant_b):
    f32 = jnp.float32
    lane = np.arange(WIDE)
    o_of = lane % GRP
    c_of = lane // GRP

    m10 = (o_of[None, :] == np.arange(OUT)[:, None]).astype(np.float32)
    mr = (lane[None, :] == (GRP * np.arange(CHUNKS) + GRP - 1)[:, None]
          ).astype(np.float32)
    real = o_of < OUT
    r_out = ((o_of[:, None] == np.arange(CHUNKS)[None, :]) & real[:, None]
             ).astype(np.float32)
    r_chunk = ((c_of[:, None] == np.arange(CHUNKS)[None, :]) & real[:, None]
               ).astype(np.float32)
    s_mat = ((c_of[:, None] == np.arange(CHUNKS)[None, :])
             & (o_of == GRP - 1)[:, None]).astype(np.float32)

    ewt2 = expert_w.transpose(2, 0, 1).reshape(OUT, IN_FEATS)
    rwt = router_w.T
    rb2 = router_b.reshape(1, CHUNKS)
    qb_t = quant_b.reshape(OUT, 1)

    def full(shape):
        return pl.BlockSpec(shape, lambda *_: tuple(0 for _ in shape))

    w, bvec = pl.pallas_call(
        _prep_kernel,
        in_specs=[
            full((OUT, IN_FEATS)),
            full((CHUNKS, IN_FEATS)),
            full((CHUNKS, OUT)),
            full((1, CHUNKS)),
            full((OUT, WIDE)),
            full((CHUNKS, WIDE)),
        ],
        out_specs=[full((IN_FEATS, WIDE)), full((1, WIDE))],
        out_shape=[
            jax.ShapeDtypeStruct((IN_FEATS, WIDE), f32),
            jax.ShapeDtypeStruct((1, WIDE), f32),
        ],
    )(ewt2, rwt, expert_b, rb2, m10, mr)

    out_pre, acts = pl.pallas_call(
        _main_kernel,
        grid=(NSTEPS,),
        in_specs=[
            pl.BlockSpec((BLK, IN_FEATS), lambda i: (i, 0)),
            full((IN_FEATS, WIDE)),
            full((1, WIDE)),
            full((WIDE, CHUNKS)),
            full((WIDE, CHUNKS)),
        ],
        out_specs=[
            pl.BlockSpec((BLK, CHUNKS), lambda i: (i, 0)),
            pl.BlockSpec((1, 1, WIDE), lambda i: (i, 0, 0)),
        ],
        out_shape=[
            jax.ShapeDtypeStruct((N_TOK, CHUNKS), f32),
            jax.ShapeDtypeStruct((NSTEPS, 1, WIDE), f32),
        ],
    )(x, w, bvec, r_out, s_mat)

    out_t = pl.pallas_call(
        _head_kernel,
        in_specs=[
            full((N_TOK, CHUNKS)),
            full((NSTEPS, 1, WIDE)),
            full((WIDE, CHUNKS)),
            full((OUT, OUT)),
            full((OUT, 1)),
        ],
        out_specs=full((OUT, N_TOK)),
        out_shape=jax.ShapeDtypeStruct((OUT, N_TOK), f32),
    )(out_pre, acts, r_chunk, quant_w, qb_t)
    return out_t.T


# speedup vs baseline: 1.3120x; 1.3120x over previous
import jax
import jax.numpy as jnp
import numpy as np
from jax.experimental import pallas as pl
from jax.experimental.pallas import tpu as pltpu

IN_FEATS = 2048
OUT = 10
CHUNKS = 16
THRESH = 0.05
CHUNK_DIM = IN_FEATS // CHUNKS
N_TOK = 16384

GRP = 16
WIDE = CHUNKS * GRP
BLK = 2048
NSTEPS = N_TOK // BLK


_DN0 = (((0,), (0,)), ((), ()))


def _prep_kernel(ew_ref, rw_ref, eb_ref, rb_ref, m10_ref, mr_ref,
                 w_ref, b_ref):
    lane = jax.lax.broadcasted_iota(jnp.int32, (1, WIDE), 1)
    rowc = jax.lax.broadcasted_iota(jnp.int32, (IN_FEATS, 1), 0) // CHUNK_DIM
    spread = jax.lax.dot_general(ew_ref[:], m10_ref[:], _DN0,
                                 preferred_element_type=jnp.float32)
    w_ref[:] = (jnp.where((lane // GRP) == rowc, spread, 0.0)
                + jax.lax.dot_general(rw_ref[:], mr_ref[:], _DN0,
                                      preferred_element_type=jnp.float32))
    ebs = jnp.dot(eb_ref[:], m10_ref[:], preferred_element_type=jnp.float32)
    crow = jax.lax.broadcasted_iota(jnp.int32, (CHUNKS, 1), 0)
    b_ref[:] = (jnp.sum(jnp.where((lane // GRP) == crow, ebs, 0.0),
                        axis=0, keepdims=True)
                + jnp.dot(rb_ref[:], mr_ref[:],
                          preferred_element_type=jnp.float32))


def _main_kernel(x_ref, w_ref, b_ref, rout_ref, outpre_ref, acts_ref):
    lane = jax.lax.broadcasted_iota(jnp.int32, (1, WIDE), 1)
    is_logit = (lane % GRP) == (GRP - 1)
    grp = lane // GRP

    acc = jnp.dot(x_ref[:], w_ref[:], preferred_element_type=jnp.float32)
    acc = acc + b_ref[:]

    lm = jnp.where(is_logit, acc, -jnp.inf)
    m = jnp.max(lm, axis=1, keepdims=True)
    e = jnp.exp(lm - m)
    s = jnp.sum(e, axis=1, keepdims=True)
    rcp = 1.0 / s
    l1 = jnp.min(jnp.where(lm == m, lane, WIDE), axis=1, keepdims=True)
    lm2 = jnp.where(lane == l1, -jnp.inf, lm)
    m2 = jnp.max(lm2, axis=1, keepdims=True)
    l2 = jnp.min(jnp.where(lm2 == m2, lane, WIDE), axis=1, keepdims=True)
    v2 = jnp.exp(m2 - m) * rcp

    gates = (jnp.where(grp == l1 // GRP, rcp, 0.0)
             + jnp.where(grp == l2 // GRP, v2, 0.0))

    outpre_ref[:] = jnp.dot(gates * acc, rout_ref[:],
                            preferred_element_type=jnp.float32)

    acts_ref[:] = jnp.sum(jnp.abs(acc), axis=0, keepdims=True)[None]


def _head_kernel(outpre_ref, acts_ref, rchunk_ref, qw_ref, qb_ref, o_ref):
    colsum = jnp.sum(acts_ref[:], axis=0)
    acts16 = jnp.dot(colsum, rchunk_ref[:],
                     preferred_element_type=jnp.float32)
    scalar_act = jnp.max(acts16) * (1.0 / (N_TOK * OUT))
    qw = jnp.pad(qw_ref[:], ((0, GRP - OUT), (0, GRP - OUT)))
    mean_abs = jnp.sum(jnp.abs(qw)) * (1.0 / (OUT * OUT))
    wq = jnp.where(scalar_act > THRESH, qw, jnp.sign(qw) * mean_abs)
    res_t = jax.lax.dot_general(wq, outpre_ref[:], (((0,), (1,)), ((), ())),
                                preferred_element_type=jnp.float32)
    res_t = res_t + jnp.pad(qb_ref[:], ((0, GRP - OUT), (0, 0)))
    o_ref[:] = res_t[:OUT, :]


def kernel(x, router_w, router_b, expert_w, expert_b, quant_w, quant_b):
    f32 = jnp.float32
    lane = np.arange(WIDE)
    o_of = lane % GRP
    c_of = lane // GRP

    m10 = (o_of[None, :] == np.arange(OUT)[:, None]).astype(np.float32)
    mr = (lane[None, :] == (GRP * np.arange(CHUNKS) + GRP - 1)[:, None]
          ).astype(np.float32)
    real = o_of < OUT
    r_out = ((o_of[:, None] == np.arange(CHUNKS)[None, :]) & real[:, None]
             ).astype(np.float32)
    r_chunk = ((c_of[:, None] == np.arange(CHUNKS)[None, :]) & real[:, None]
               ).astype(np.float32)

    ewt2 = expert_w.transpose(2, 0, 1).reshape(OUT, IN_FEATS)
    rwt = router_w.T
    rb2 = router_b.reshape(1, CHUNKS)
    qb_t = quant_b.reshape(OUT, 1)

    def full(shape):
        return pl.BlockSpec(shape, lambda *_: tuple(0 for _ in shape))

    w, bvec = pl.pallas_call(
        _prep_kernel,
        in_specs=[
            full((OUT, IN_FEATS)),
            full((CHUNKS, IN_FEATS)),
            full((CHUNKS, OUT)),
            full((1, CHUNKS)),
            full((OUT, WIDE)),
            full((CHUNKS, WIDE)),
        ],
        out_specs=[full((IN_FEATS, WIDE)), full((1, WIDE))],
        out_shape=[
            jax.ShapeDtypeStruct((IN_FEATS, WIDE), f32),
            jax.ShapeDtypeStruct((1, WIDE), f32),
        ],
    )(ewt2, rwt, expert_b, rb2, m10, mr)

    out_pre, acts = pl.pallas_call(
        _main_kernel,
        grid=(NSTEPS,),
        in_specs=[
            pl.BlockSpec((BLK, IN_FEATS), lambda i: (i, 0)),
            full((IN_FEATS, WIDE)),
            full((1, WIDE)),
            full((WIDE, CHUNKS)),
        ],
        out_specs=[
            pl.BlockSpec((BLK, CHUNKS), lambda i: (i, 0)),
            pl.BlockSpec((1, 1, WIDE), lambda i: (i, 0, 0)),
        ],
        out_shape=[
            jax.ShapeDtypeStruct((N_TOK, CHUNKS), f32),
            jax.ShapeDtypeStruct((NSTEPS, 1, WIDE), f32),
        ],
    )(x, w, bvec, r_out)

    out_t = pl.pallas_call(
        _head_kernel,
        in_specs=[
            full((N_TOK, CHUNKS)),
            full((NSTEPS, 1, WIDE)),
            full((WIDE, CHUNKS)),
            full((OUT, OUT)),
            full((OUT, 1)),
        ],
        out_specs=full((OUT, N_TOK)),
        out_shape=jax.ShapeDtypeStruct((OUT, N_TOK), f32),
    )(out_pre, acts, r_chunk, quant_w, qb_t)
    return out_t.T


# speedup vs baseline: 1.3189x; 1.0052x over previous
import jax
import jax.numpy as jnp
import numpy as np
from jax.experimental import pallas as pl

IN_FEATS = 2048
OUT = 10
CHUNKS = 16
THRESH = 0.05
CHUNK_DIM = IN_FEATS // CHUNKS
N_TOK = 16384

GRP = 16
WIDE = CHUNKS * GRP
BLK = 2048
NSTEPS = N_TOK // BLK


_DN0 = (((0,), (0,)), ((), ()))


def _prep_kernel(ew_ref, rw_ref, eb_ref, rb_ref, m10_ref, mr_ref,
                 w_ref, b_ref):
    lane = jax.lax.broadcasted_iota(jnp.int32, (1, WIDE), 1)
    rowc = jax.lax.broadcasted_iota(jnp.int32, (IN_FEATS, 1), 0) // CHUNK_DIM
    spread = jax.lax.dot_general(ew_ref[:], m10_ref[:], _DN0,
                                 preferred_element_type=jnp.float32)
    w_ref[:] = (jnp.where((lane // GRP) == rowc, spread, 0.0)
                + jax.lax.dot_general(rw_ref[:], mr_ref[:], _DN0,
                                      preferred_element_type=jnp.float32))
    ebs = jnp.dot(eb_ref[:], m10_ref[:], preferred_element_type=jnp.float32)
    crow = jax.lax.broadcasted_iota(jnp.int32, (CHUNKS, 1), 0)
    b_ref[:] = (jnp.sum(jnp.where((lane // GRP) == crow, ebs, 0.0),
                        axis=0, keepdims=True)
                + jnp.dot(rb_ref[:], mr_ref[:],
                          preferred_element_type=jnp.float32))


def _main_kernel(x_ref, w_ref, b_ref, rout_ref, outpre_ref, acts_ref):
    lane = jax.lax.broadcasted_iota(jnp.int32, (1, WIDE), 1)
    is_logit = (lane % GRP) == (GRP - 1)
    grp = lane // GRP

    acc = jnp.dot(x_ref[:], w_ref[:], preferred_element_type=jnp.float32)
    acc = acc + b_ref[:]

    lm = jnp.where(is_logit, acc, -jnp.inf)
    m = jnp.max(lm, axis=1, keepdims=True)
    e = jnp.exp(lm - m)
    s = jnp.sum(e, axis=1, keepdims=True)
    rcp = 1.0 / s
    l1 = jnp.min(jnp.where(lm == m, lane, WIDE), axis=1, keepdims=True)
    lm2 = jnp.where(lane == l1, -jnp.inf, lm)
    m2 = jnp.max(lm2, axis=1, keepdims=True)
    l2 = jnp.min(jnp.where(lm2 == m2, lane, WIDE), axis=1, keepdims=True)
    v2 = jnp.exp(m2 - m) * rcp

    gates = (jnp.where(grp == l1 // GRP, rcp, 0.0)
             + jnp.where(grp == l2 // GRP, v2, 0.0))

    outpre_ref[:] = jnp.dot(gates * acc, rout_ref[:],
                            preferred_element_type=jnp.float32)

    acts_ref[:] = jnp.sum(jnp.abs(acc), axis=0, keepdims=True)[None]


def _head_kernel(outpre_ref, acts_ref, rchunk_ref, qw_ref, qb_ref, o_ref):
    colsum = jnp.sum(acts_ref[:], axis=0)
    acts16 = jnp.dot(colsum, rchunk_ref[:],
                     preferred_element_type=jnp.float32)
    scalar_act = jnp.max(acts16) * (1.0 / (N_TOK * OUT))
    qw = jnp.pad(qw_ref[:], ((0, GRP - OUT), (0, GRP - OUT)))
    mean_abs = jnp.sum(jnp.abs(qw)) * (1.0 / (OUT * OUT))
    wq = jnp.where(scalar_act > THRESH, qw, jnp.sign(qw) * mean_abs)
    res_t = jax.lax.dot_general(wq, outpre_ref[:], (((0,), (1,)), ((), ())),
                                preferred_element_type=jnp.float32)
    res_t = res_t + jnp.pad(qb_ref[:], ((0, GRP - OUT), (0, 0)))
    o_ref[:] = res_t[:OUT, :]


def kernel(x, router_w, router_b, expert_w, expert_b, quant_w, quant_b):
    f32 = jnp.float32
    lane = np.arange(WIDE)
    o_of = lane % GRP
    c_of = lane // GRP

    m10 = (o_of[None, :] == np.arange(OUT)[:, None]).astype(np.float32)
    mr = (lane[None, :] == (GRP * np.arange(CHUNKS) + GRP - 1)[:, None]
          ).astype(np.float32)
    real = o_of < OUT
    r_out = ((o_of[:, None] == np.arange(CHUNKS)[None, :]) & real[:, None]
             ).astype(np.float32)
    r_chunk = ((c_of[:, None] == np.arange(CHUNKS)[None, :]) & real[:, None]
               ).astype(np.float32)

    ewt2 = expert_w.transpose(2, 0, 1).reshape(OUT, IN_FEATS)
    rwt = router_w.T
    rb2 = router_b.reshape(1, CHUNKS)
    qb_t = quant_b.reshape(OUT, 1)

    def full(shape):
        return pl.BlockSpec(shape, lambda *_: tuple(0 for _ in shape))

    w, bvec = pl.pallas_call(
        _prep_kernel,
        in_specs=[
            full((OUT, IN_FEATS)),
            full((CHUNKS, IN_FEATS)),
            full((CHUNKS, OUT)),
            full((1, CHUNKS)),
            full((OUT, WIDE)),
            full((CHUNKS, WIDE)),
        ],
        out_specs=[full((IN_FEATS, WIDE)), full((1, WIDE))],
        out_shape=[
            jax.ShapeDtypeStruct((IN_FEATS, WIDE), f32),
            jax.ShapeDtypeStruct((1, WIDE), f32),
        ],
    )(ewt2, rwt, expert_b, rb2, m10, mr)

    out_pre, acts = pl.pallas_call(
        _main_kernel,
        grid=(NSTEPS,),
        in_specs=[
            pl.BlockSpec((BLK, IN_FEATS), lambda i: (i, 0)),
            full((IN_FEATS, WIDE)),
            full((1, WIDE)),
            full((WIDE, CHUNKS)),
        ],
        out_specs=[
            pl.BlockSpec((BLK, CHUNKS), lambda i: (i, 0)),
            pl.BlockSpec((1, 1, WIDE), lambda i: (i, 0, 0)),
        ],
        out_shape=[
            jax.ShapeDtypeStruct((N_TOK, CHUNKS), f32),
            jax.ShapeDtypeStruct((NSTEPS, 1, WIDE), f32),
        ],
    )(x, w, bvec, r_out)

    out_t = pl.pallas_call(
        _head_kernel,
        in_specs=[
            full((N_TOK, CHUNKS)),
            full((NSTEPS, 1, WIDE)),
            full((WIDE, CHUNKS)),
            full((OUT, OUT)),
            full((OUT, 1)),
        ],
        out_specs=full((OUT, N_TOK)),
        out_shape=jax.ShapeDtypeStruct((OUT, N_TOK), f32),
    )(out_pre, acts, r_chunk, quant_w, qb_t)
    return out_t.T
